# baseline (speedup 1.0000x reference)
import jax
import jax.numpy as jnp
from jax.experimental import pallas as pl
from jax.experimental.pallas import tpu as pltpu

B, SEQ, D = 32, 196, 768
POOL, LENGTH, TOPK = 100, 10, 5
RB = 8


def _fused_kernel(x_ref, x_key_ref, prompt_ref, prompt_key_ref,
                  out_ref, rs_ref):
    j = pl.program_id(0)

    out_ref[:, LENGTH:, :] = x_ref[...]

    xk = x_key_ref[pl.ds(j * RB, RB), :]
    xk = xk / jnp.maximum(
        jnp.sqrt(jnp.sum(xk * xk, axis=1, keepdims=True)), 1e-12)
    pk = prompt_key_ref[...]
    pk = pk / jnp.maximum(
        jnp.sqrt(jnp.sum(pk * pk, axis=1, keepdims=True)), 1e-12)

    sim = jnp.dot(xk, pk.T, preferred_element_type=jnp.float32)
    m = jnp.max(sim, axis=1, keepdims=True)
    e = jnp.exp(sim - m)
    p = e / jnp.sum(e, axis=1, keepdims=True)

    for l in range(LENGTH):
        out_ref[:, l, :] = jnp.dot(
            p, prompt_ref[:, l, :],
            preferred_element_type=jnp.float32) * (1.0 / POOL)

    @pl.when(j == 0)
    def _topk():
        xka = x_key_ref[...]
        xka = xka / jnp.maximum(
            jnp.sqrt(jnp.sum(xka * xka, axis=1, keepdims=True)), 1e-12)
        sima = jnp.dot(xka, pk.T, preferred_element_type=jnp.float32)
        iota = jax.lax.broadcasted_iota(jnp.int32, (B, POOL), 1)
        v = sima
        total = jnp.float32(0.0)
        for _ in range(TOPK):
            mx = jnp.max(v, axis=1, keepdims=True)
            idx = jnp.min(jnp.where(v >= mx, iota, jnp.int32(POOL)),
                          axis=1, keepdims=True)
            total = total + jnp.sum(mx)
            v = jnp.where(iota == idx, -jnp.inf, v)
        rs_ref[...] = jnp.full((1, 1), total * (1.0 / B), jnp.float32)


@jax.jit
def kernel(x_embed, x_key, prompt, prompt_key):
    out, rs = pl.pallas_call(
        _fused_kernel,
        grid=(B // RB,),
        in_specs=[
            pl.BlockSpec((RB, SEQ, D), lambda j: (j, 0, 0)),
            pl.BlockSpec((B, 2 * D), lambda j: (0, 0)),
            pl.BlockSpec((POOL, LENGTH, D), lambda j: (0, 0, 0)),
            pl.BlockSpec((POOL, 2 * D), lambda j: (0, 0)),
        ],
        out_specs=[
            pl.BlockSpec((RB, LENGTH + SEQ, D), lambda j: (j, 0, 0)),
            pl.BlockSpec((1, 1), lambda j: (0, 0)),
        ],
        out_shape=[
            jax.ShapeDtypeStruct((B, LENGTH + SEQ, D), jnp.float32),
            jax.ShapeDtypeStruct((1, 1), jnp.float32),
        ],
        compiler_params=pltpu.CompilerParams(
            dimension_semantics=("parallel",),
        ),
    )(x_embed, x_key, prompt, prompt_key)
    return out, rs[0, 0]
